# baseline (speedup 1.0000x reference)
import functools

import jax
import jax.numpy as jnp
from jax import lax
from jax.experimental import pallas as pl
from jax.experimental.pallas import tpu as pltpu
from jax.experimental.pallas import tpu_sc as plsc

U = 30000
I = 20000
N = U + I
D = 64
E = 800000
GAMMA = 0.5
N_LAYERS = 2

BM_I = 1000
BM_U = 1000


def _ets_block(x1, x2, W1t, b1, w2):
    q1 = jnp.dot(jnp.tanh(jnp.dot(x1, W1t, preferred_element_type=jnp.float32) + b1), w2,
                 preferred_element_type=jnp.float32)
    q2 = jnp.dot(jnp.tanh(jnp.dot(x2, W1t, preferred_element_type=jnp.float32) + b1), w2,
                 preferred_element_type=jnp.float32)
    w0 = jax.nn.sigmoid(q1 - q2)[:, None]
    return w0 * x1 + (1.0 - w0) * x2


def _normalize_block(x):
    n = jnp.sqrt(jnp.sum(x * x, axis=1, keepdims=True))
    return x / jnp.maximum(n, 1e-12)


def _store_split(out_ref, x):
    out_ref[0] = x[:, :32]
    out_ref[1] = x[:, 32:]


def _load_split(ref):
    return jnp.concatenate([ref[0], ref[1]], axis=1)


def _items_dense_kernel(imgf_ref, txtf_ref, itid_ref, iiid_ref,
                        i2iWt_ref, i2ib_ref, t2iWt_ref, t2ib_ref,
                        iiW1t_ref, iib1_ref, iiw2_ref,
                        tiW1t_ref, tib1_ref, tiw2_ref,
                        fiW1t_ref, fib1_ref, fiw2_ref,
                        image_out, text_out, isi_out, ist_out, ife_out,
                        imgn_out, txtn_out, guii_out, guti_out):
    image = jnp.dot(imgf_ref[...], i2iWt_ref[...],
                    preferred_element_type=jnp.float32) + i2ib_ref[...]
    text = jnp.dot(txtf_ref[...], t2iWt_ref[...],
                   preferred_element_type=jnp.float32) + t2ib_ref[...]
    isi = _ets_block(image, iiid_ref[...], iiW1t_ref[...], iib1_ref[...],
                     iiw2_ref[0, :])
    ist = _ets_block(text, itid_ref[...], tiW1t_ref[...], tib1_ref[...],
                     tiw2_ref[0, :])
    ife = _ets_block(isi, ist, fiW1t_ref[...], fib1_ref[...], fiw2_ref[0, :])
    image_out[...] = image
    text_out[...] = text
    isi_out[...] = isi
    ist_out[...] = ist
    ife_out[...] = ife
    _store_split(imgn_out, _normalize_block(image))
    _store_split(txtn_out, _normalize_block(text))
    _store_split(guii_out, GAMMA * iiid_ref[...])
    _store_split(guti_out, GAMMA * itid_ref[...])


def _users_norm_kernel(uimg_ref, utxt_ref, uid_ref,
                       uimgn_out, utxtn_out, guu_out):
    _store_split(uimgn_out, _normalize_block(uimg_ref[...]))
    _store_split(utxtn_out, _normalize_block(utxt_ref[...]))
    _store_split(guu_out, GAMMA * uid_ref[...])


def _final_users_kernel(t0_ref, t1_ref, t2_ref, g0_ref, g1_ref, g2_ref,
                        W1t_ref, b1_ref, w2_ref, ug_out):
    all_txt = (_load_split(t0_ref) + _load_split(t1_ref)
               + _load_split(t2_ref)) * (1.0 / 3.0)
    all_img = (_load_split(g0_ref) + _load_split(g1_ref)
               + _load_split(g2_ref)) * (1.0 / 3.0)
    ug_out[...] = _ets_block(all_txt, all_img, W1t_ref[...], b1_ref[...],
                             w2_ref[0, :])


def _final_items_kernel(t0_ref, t1_ref, t2_ref, g0_ref, g1_ref, g2_ref,
                        ife_ref, W1t_ref, b1_ref, w2_ref, ig_out):
    all_txt = (_load_split(t0_ref) + _load_split(t1_ref)
               + _load_split(t2_ref)) * (1.0 / 3.0)
    all_img = (_load_split(g0_ref) + _load_split(g1_ref)
               + _load_split(g2_ref)) * (1.0 / 3.0)
    ig_out[...] = (_ets_block(all_txt, all_img, W1t_ref[...], b1_ref[...],
                              w2_ref[0, :])
                   + _normalize_block(ife_ref[...]))


def _row_spec(bm):
    return pl.BlockSpec((bm, D), lambda i: (i, 0))


def _full_spec(shape):
    return pl.BlockSpec(shape, lambda i: tuple(0 for _ in shape))


NSUB = 16
HD = D // 2
KE = 128
NP = 50048
RPW = NP // NSUB
NCHUNK = 394
EPW = NCHUNK * KE
EP = EPW * NSUB


def _spmm2_sc_body(adj_hbm, vals_hbm, x0i_hbm, gui_hbm, x0t_hbm, gut_hbm,
                   e1i_hbm, e2i_hbm, e1t_hbm, e2t_hbm,
                   acc, eidx, vv, xr,
                   gs0, gs1, ss0, ss1, es0, es1, es2):
    cid = lax.axis_index("c")
    sid = lax.axis_index("s")
    rlo = sid * RPW
    gs = (gs0, gs1)
    ss = (ss0, ss1)
    es = (es0, es1, es2)

    def init_acc(gu_hbm):
        pltpu.sync_copy(gu_hbm.at[cid, pl.ds(rlo, RPW)],
                        acc.at[pl.ds(rlo, RPW)])

    def edge_pass(src_hbm):
        src = src_hbm.at[cid]
        ebase = sid * EPW

        def issue_edata(c, b3):
            pltpu.async_copy(adj_hbm.at[:, pl.ds(ebase + c * KE, KE)],
                             eidx.at[b3], es[b3])
            pltpu.async_copy(vals_hbm.at[pl.ds(ebase + c * KE, KE)],
                             vv.at[b3], es[b3])

        def wait_edata(b3):
            pltpu.make_async_copy(adj_hbm.at[:, pl.ds(0, KE)],
                                  eidx.at[b3], es[b3]).wait()
            pltpu.make_async_copy(vals_hbm.at[pl.ds(0, KE)],
                                  vv.at[b3], es[b3]).wait()

        def issue_gather(b3, b2):
            pltpu.async_copy(src.at[eidx.at[b3].at[1]], xr.at[b2], gs[b2])

        def wait_gather(b2):
            pltpu.make_async_copy(src.at[eidx.at[0].at[1]],
                                  xr.at[b2], gs[b2]).wait()

        def issue_scatter(b3, b2):
            pltpu.async_copy(xr.at[b2], acc.at[eidx.at[b3].at[0]], ss[b2],
                             add=True)

        def wait_scatter(b2):
            pltpu.make_async_copy(xr.at[b2], acc.at[eidx.at[0].at[0]],
                                  ss[b2]).wait()

        def scale(b3, b2):
            def grp(g16, _):
                w = vv.at[b3][pl.ds(g16 * 16, 16)]
                e0 = g16 * 16
                for j in range(16):
                    v = w[j]
                    xr[b2, e0 + j, pl.ds(0, 16)] = \
                        xr[b2, e0 + j, pl.ds(0, 16)] * v
                    xr[b2, e0 + j, pl.ds(16, 16)] = \
                        xr[b2, e0 + j, pl.ds(16, 16)] * v
                return 0

            lax.fori_loop(0, KE // 16, grp, 0)

        def slot(c, b2, b3, first=False, do_next_gather=True,
                 do_edata=True):
            nb2, nb3 = 1 - b2, (b3 + 1) % 3
            if do_next_gather:
                wait_edata(nb3)
                if not first:
                    wait_scatter(nb2)
                issue_gather(nb3, nb2)
            if do_edata:
                issue_edata(c + 2, (b3 + 2) % 3)
            wait_gather(b2)
            scale(b3, b2)
            issue_scatter(b3, b2)

        issue_edata(0, 0)
        issue_edata(1, 1)
        wait_edata(0)
        issue_gather(0, 0)
        slot(0, 0, 0, first=True)
        slot(1, 1, 1)

        def body(k, _):
            c0 = 2 + 6 * k
            for j in range(6):
                slot(c0 + j, j % 2, (2 + j) % 3)
            return 0

        lax.fori_loop(0, (NCHUNK - 4) // 6, body, 0)
        slot(NCHUNK - 2, 0, (NCHUNK - 2) % 3, do_edata=False)
        slot(NCHUNK - 1, 1, (NCHUNK - 1) % 3, do_next_gather=False,
             do_edata=False)
        wait_scatter(0)
        wait_scatter(1)

    def chain(x0_hbm, gu_hbm, e1_hbm, e2_hbm):
        init_acc(gu_hbm)
        plsc.subcore_barrier()
        edge_pass(x0_hbm)
        plsc.subcore_barrier()
        pltpu.sync_copy(acc.at[pl.ds(rlo, RPW)],
                        e1_hbm.at[cid, pl.ds(rlo, RPW)])
        plsc.subcore_barrier()
        init_acc(gu_hbm)
        plsc.subcore_barrier()
        edge_pass(e1_hbm)
        plsc.subcore_barrier()
        pltpu.sync_copy(acc.at[pl.ds(rlo, RPW)],
                        e2_hbm.at[cid, pl.ds(rlo, RPW)])

    chain(x0i_hbm, gui_hbm, e1i_hbm, e2i_hbm)
    plsc.subcore_barrier()
    chain(x0t_hbm, gut_hbm, e1t_hbm, e2t_hbm)


@functools.lru_cache(maxsize=1)
def _get_spmm2_sc():
    return pl.kernel(
        _spmm2_sc_body,
        mesh=plsc.VectorSubcoreMesh(core_axis_name="c",
                                    subcore_axis_name="s"),
        compiler_params=pltpu.CompilerParams(use_tc_tiling_on_sc=False),
        out_type=[jax.ShapeDtypeStruct((2, NP, HD), jnp.float32)] * 4,
        scratch_types=[
            pltpu.VMEM_SHARED((NP, HD), jnp.float32),
            pltpu.VMEM((3, 2, KE), jnp.int32),
            pltpu.VMEM((3, KE), jnp.float32),
            pltpu.VMEM((2, KE, HD), jnp.float32),
        ] + [pltpu.SemaphoreType.DMA] * 7,
    )


def _spmm2_sc(*args):
    return _get_spmm2_sc()(*args)


def _pad_edges(adj_indices, adj_values):
    adj_p = jnp.pad(adj_indices, ((0, 0), (0, EP - E)))
    vals_p = jnp.pad(adj_values, (0, EP - E))
    return adj_p, vals_p


def _split_spec(bm):
    return pl.BlockSpec((2, bm, HD), lambda i: (0, i, 0))


def _split_spec_off(bm, off):
    return pl.BlockSpec((2, bm, HD), lambda i: (0, i + off, 0))


def kernel(adj_indices, adj_values, user_id, item_tid, item_iid, upref_img,
           upref_txt, image_feats, text_feats, t2i_W, t2i_b, i2i_W, i2i_b,
           ti_W1, ti_b1, ti_w2, ii_W1, ii_b1, ii_w2, fi_W1, fi_b1, fi_w2,
           uf_W1, uf_b1, uf_w2, itf_W1, itf_b1, itf_w2):
    f32 = jnp.float32
    vec = lambda v: v.reshape(1, D)

    n_i = I // BM_I
    items_out = pl.pallas_call(
        _items_dense_kernel,
        grid=(n_i,),
        in_specs=[
            pl.BlockSpec((BM_I, 4096), lambda i: (i, 0)),
            pl.BlockSpec((BM_I, 768), lambda i: (i, 0)),
            _row_spec(BM_I), _row_spec(BM_I),
            _full_spec((4096, D)), _full_spec((1, D)),
            _full_spec((768, D)), _full_spec((1, D)),
            _full_spec((D, D)), _full_spec((1, D)), _full_spec((1, D)),
            _full_spec((D, D)), _full_spec((1, D)), _full_spec((1, D)),
            _full_spec((D, D)), _full_spec((1, D)), _full_spec((1, D)),
        ],
        out_specs=[_row_spec(BM_I)] * 5 + [_split_spec(BM_I)] * 4,
        out_shape=[jax.ShapeDtypeStruct((I, D), f32)] * 5
        + [jax.ShapeDtypeStruct((2, I, HD), f32)] * 4,
    )(image_feats, text_feats, item_tid, item_iid,
      i2i_W.T, vec(i2i_b), t2i_W.T, vec(t2i_b),
      ii_W1.T, vec(ii_b1), vec(ii_w2),
      ti_W1.T, vec(ti_b1), vec(ti_w2),
      fi_W1.T, vec(fi_b1), vec(fi_w2))
    image, text, isi, ist, ife, imgn_s, txtn_s, guii_s, guti_s = items_out

    n_u = U // BM_U
    uimgn_s, utxtn_s, guu_s = pl.pallas_call(
        _users_norm_kernel,
        grid=(n_u,),
        in_specs=[_row_spec(BM_U)] * 3,
        out_specs=[_split_spec(BM_U)] * 3,
        out_shape=[jax.ShapeDtypeStruct((2, U, HD), f32)] * 3,
    )(upref_img, upref_txt, user_id)

    pad_z = jnp.zeros((2, NP - N, HD), f32)
    x0_img = jnp.concatenate([uimgn_s, imgn_s, pad_z], axis=1)
    x0_txt = jnp.concatenate([utxtn_s, txtn_s, pad_z], axis=1)
    gu_ii = jnp.concatenate([guu_s, guii_s, pad_z], axis=1)
    gu_ti = jnp.concatenate([guu_s, guti_s, pad_z], axis=1)

    adj_p, vals_p = _pad_edges(adj_indices, adj_values)
    img1, img2, txt1, txt2 = _spmm2_sc(adj_p, vals_p, x0_img, gu_ii,
                                       x0_txt, gu_ti)

    u_g = pl.pallas_call(
        _final_users_kernel,
        grid=(n_u,),
        in_specs=[_split_spec(BM_U)] * 6 + [
            _full_spec((D, D)), _full_spec((1, D)), _full_spec((1, D))],
        out_specs=_row_spec(BM_U),
        out_shape=jax.ShapeDtypeStruct((U, D), f32),
    )(x0_txt, txt1, txt2, x0_img, img1, img2,
      uf_W1.T, vec(uf_b1), vec(uf_w2))

    i_g = pl.pallas_call(
        _final_items_kernel,
        grid=(n_i,),
        in_specs=[_split_spec_off(BM_I, U // BM_I)] * 6 + [
            _row_spec(BM_I),
            _full_spec((D, D)), _full_spec((1, D)), _full_spec((1, D))],
        out_specs=_row_spec(BM_I),
        out_shape=jax.ShapeDtypeStruct((I, D), f32),
    )(x0_txt, txt1, txt2, x0_img, img1, img2,
      ife, itf_W1.T, vec(itf_b1), vec(itf_w2))

    return (u_g, i_g, image, item_iid, text, item_tid, isi, ist, ife)


# speedup vs baseline: 1.0503x; 1.0503x over previous
import functools

import jax
import jax.numpy as jnp
from jax import lax
from jax.experimental import pallas as pl
from jax.experimental.pallas import tpu as pltpu
from jax.experimental.pallas import tpu_sc as plsc

U = 30000
I = 20000
N = U + I
D = 64
E = 800000
GAMMA = 0.5
N_LAYERS = 2

BM_I = 1000
BM_U = 1000


def _ets_block(x1, x2, W1t, b1, w2):
    q1 = jnp.dot(jnp.tanh(jnp.dot(x1, W1t, preferred_element_type=jnp.float32) + b1), w2,
                 preferred_element_type=jnp.float32)
    q2 = jnp.dot(jnp.tanh(jnp.dot(x2, W1t, preferred_element_type=jnp.float32) + b1), w2,
                 preferred_element_type=jnp.float32)
    w0 = jax.nn.sigmoid(q1 - q2)[:, None]
    return w0 * x1 + (1.0 - w0) * x2


def _normalize_block(x):
    n = jnp.sqrt(jnp.sum(x * x, axis=1, keepdims=True))
    return x / jnp.maximum(n, 1e-12)


def _store_split(out_ref, x):
    out_ref[0] = x[:, :32]
    out_ref[1] = x[:, 32:]


def _load_split(ref):
    return jnp.concatenate([ref[0], ref[1]], axis=1)


def _items_dense_kernel(imgf_ref, txtf_ref, itid_ref, iiid_ref,
                        i2iWt_ref, i2ib_ref, t2iWt_ref, t2ib_ref,
                        iiW1t_ref, iib1_ref, iiw2_ref,
                        tiW1t_ref, tib1_ref, tiw2_ref,
                        fiW1t_ref, fib1_ref, fiw2_ref,
                        image_out, text_out, isi_out, ist_out, ife_out,
                        imgn_out, txtn_out, guii_out, guti_out):
    image = jnp.dot(imgf_ref[...], i2iWt_ref[...],
                    preferred_element_type=jnp.float32) + i2ib_ref[...]
    text = jnp.dot(txtf_ref[...], t2iWt_ref[...],
                   preferred_element_type=jnp.float32) + t2ib_ref[...]
    isi = _ets_block(image, iiid_ref[...], iiW1t_ref[...], iib1_ref[...],
                     iiw2_ref[0, :])
    ist = _ets_block(text, itid_ref[...], tiW1t_ref[...], tib1_ref[...],
                     tiw2_ref[0, :])
    ife = _ets_block(isi, ist, fiW1t_ref[...], fib1_ref[...], fiw2_ref[0, :])
    image_out[...] = image
    text_out[...] = text
    isi_out[...] = isi
    ist_out[...] = ist
    ife_out[...] = ife
    _store_split(imgn_out, _normalize_block(image))
    _store_split(txtn_out, _normalize_block(text))
    _store_split(guii_out, GAMMA * iiid_ref[...])
    _store_split(guti_out, GAMMA * itid_ref[...])


def _users_norm_kernel(uimg_ref, utxt_ref, uid_ref,
                       uimgn_out, utxtn_out, guu_out):
    _store_split(uimgn_out, _normalize_block(uimg_ref[...]))
    _store_split(utxtn_out, _normalize_block(utxt_ref[...]))
    _store_split(guu_out, GAMMA * uid_ref[...])


def _final_users_kernel(ts_ref, gs_ref, W1t_ref, b1_ref, w2_ref, ug_out):
    all_txt = _load_split(ts_ref) * (1.0 / 3.0)
    all_img = _load_split(gs_ref) * (1.0 / 3.0)
    ug_out[...] = _ets_block(all_txt, all_img, W1t_ref[...], b1_ref[...],
                             w2_ref[0, :])


def _final_items_kernel(ts_ref, gs_ref, ife_ref, W1t_ref, b1_ref, w2_ref,
                        ig_out):
    all_txt = _load_split(ts_ref) * (1.0 / 3.0)
    all_img = _load_split(gs_ref) * (1.0 / 3.0)
    ig_out[...] = (_ets_block(all_txt, all_img, W1t_ref[...], b1_ref[...],
                              w2_ref[0, :])
                   + _normalize_block(ife_ref[...]))


def _row_spec(bm):
    return pl.BlockSpec((bm, D), lambda i: (i, 0))


def _full_spec(shape):
    return pl.BlockSpec(shape, lambda i: tuple(0 for _ in shape))


NSUB = 16
HD = D // 2
KE = 128
NP = 50048
RPW = NP // NSUB
NCHUNK = 394
EPW = NCHUNK * KE
EP = EPW * NSUB


CKR = 128
CKT = RPW - (RPW // CKR) * CKR


def _spmm2_sc_body(adj_hbm, vals_hbm, x0_hbm, gu_hbm, e1_hbm, esum_hbm,
                   acc, eidx, vv, xr, ts,
                   gs0, gs1, ss0, ss1, es0, es1, es2):
    cid = lax.axis_index("c")
    sid = lax.axis_index("s")
    rlo = sid * RPW
    gs = (gs0, gs1)
    ss = (ss0, ss1)
    es = (es0, es1, es2)

    def init_acc(gu_hbm):
        pltpu.sync_copy(gu_hbm.at[cid, pl.ds(rlo, RPW)],
                        acc.at[pl.ds(rlo, RPW)])

    def edge_pass(src_hbm):
        src = src_hbm.at[cid]
        ebase = sid * EPW

        def issue_edata(c, b3):
            pltpu.async_copy(adj_hbm.at[:, pl.ds(ebase + c * KE, KE)],
                             eidx.at[b3], es[b3])
            pltpu.async_copy(vals_hbm.at[pl.ds(ebase + c * KE, KE)],
                             vv.at[b3], es[b3])

        def wait_edata(b3):
            pltpu.make_async_copy(adj_hbm.at[:, pl.ds(0, KE)],
                                  eidx.at[b3], es[b3]).wait()
            pltpu.make_async_copy(vals_hbm.at[pl.ds(0, KE)],
                                  vv.at[b3], es[b3]).wait()

        def issue_gather(b3, b2):
            pltpu.async_copy(src.at[eidx.at[b3].at[1]], xr.at[b2], gs[b2])

        def wait_gather(b2):
            pltpu.make_async_copy(src.at[eidx.at[0].at[1]],
                                  xr.at[b2], gs[b2]).wait()

        def issue_scatter(b3, b2):
            pltpu.async_copy(xr.at[b2], acc.at[eidx.at[b3].at[0]], ss[b2],
                             add=True)

        def wait_scatter(b2):
            pltpu.make_async_copy(xr.at[b2], acc.at[eidx.at[0].at[0]],
                                  ss[b2]).wait()

        def scale(b3, b2):
            def grp(g16, _):
                w = vv.at[b3][pl.ds(g16 * 16, 16)]
                e0 = g16 * 16
                for j in range(16):
                    v = w[j]
                    xr[b2, e0 + j, pl.ds(0, 16)] = \
                        xr[b2, e0 + j, pl.ds(0, 16)] * v
                    xr[b2, e0 + j, pl.ds(16, 16)] = \
                        xr[b2, e0 + j, pl.ds(16, 16)] * v
                return 0

            lax.fori_loop(0, KE // 16, grp, 0)

        def slot(c, b2, b3, first=False, do_next_gather=True,
                 do_edata=True):
            nb2, nb3 = 1 - b2, (b3 + 1) % 3
            if do_next_gather:
                wait_edata(nb3)
                if not first:
                    wait_scatter(nb2)
                issue_gather(nb3, nb2)
            if do_edata:
                issue_edata(c + 2, (b3 + 2) % 3)
            wait_gather(b2)
            scale(b3, b2)
            issue_scatter(b3, b2)

        issue_edata(0, 0)
        issue_edata(1, 1)
        wait_edata(0)
        issue_gather(0, 0)
        slot(0, 0, 0, first=True)
        slot(1, 1, 1)

        def body(k, _):
            c0 = 2 + 6 * k
            for j in range(6):
                slot(c0 + j, j % 2, (2 + j) % 3)
            return 0

        lax.fori_loop(0, (NCHUNK - 4) // 6, body, 0)
        slot(NCHUNK - 2, 0, (NCHUNK - 2) % 3, do_edata=False)
        slot(NCHUNK - 1, 1, (NCHUNK - 1) % 3, do_next_gather=False,
             do_edata=False)
        wait_scatter(0)
        wait_scatter(1)

    def layer_sum():
        def piece(row0, sz):
            pltpu.sync_copy(x0_hbm.at[cid, pl.ds(row0, sz)],
                            xr.at[0].at[pl.ds(0, sz)])
            pltpu.sync_copy(e1_hbm.at[cid, pl.ds(row0, sz)],
                            xr.at[1].at[pl.ds(0, sz)])
            pltpu.sync_copy(acc.at[pl.ds(row0, sz)], ts.at[pl.ds(0, sz)])

            def addrow(r, _):
                for h in range(2):
                    sl = pl.ds(h * 16, 16)
                    ts[r, sl] = ts[r, sl] + xr[0, r, sl] + xr[1, r, sl]
                return 0

            lax.fori_loop(0, sz, addrow, 0)
            pltpu.sync_copy(ts.at[pl.ds(0, sz)],
                            esum_hbm.at[cid, pl.ds(row0, sz)])

        def fullpiece(k, _):
            piece(rlo + k * CKR, CKR)
            return 0

        lax.fori_loop(0, RPW // CKR, fullpiece, 0)
        piece(rlo + (RPW // CKR) * CKR, CKT)

    init_acc(gu_hbm)
    plsc.subcore_barrier()
    edge_pass(x0_hbm)
    plsc.subcore_barrier()
    pltpu.sync_copy(acc.at[pl.ds(rlo, RPW)],
                    e1_hbm.at[cid, pl.ds(rlo, RPW)])
    plsc.subcore_barrier()
    init_acc(gu_hbm)
    plsc.subcore_barrier()
    edge_pass(e1_hbm)
    plsc.subcore_barrier()
    layer_sum()


@functools.lru_cache(maxsize=1)
def _get_spmm2_sc():
    return pl.kernel(
        _spmm2_sc_body,
        mesh=plsc.VectorSubcoreMesh(core_axis_name="c",
                                    subcore_axis_name="s"),
        compiler_params=pltpu.CompilerParams(use_tc_tiling_on_sc=False),
        out_type=[jax.ShapeDtypeStruct((2, NP, HD), jnp.float32)] * 2,
        scratch_types=[
            pltpu.VMEM_SHARED((NP, HD), jnp.float32),
            pltpu.VMEM((3, 2, KE), jnp.int32),
            pltpu.VMEM((3, KE), jnp.float32),
            pltpu.VMEM((2, KE, HD), jnp.float32),
            pltpu.VMEM((CKR, HD), jnp.float32),
        ] + [pltpu.SemaphoreType.DMA] * 7,
    )


def _spmm2_sc(*args):
    return _get_spmm2_sc()(*args)


def _pad_edges(adj_indices, adj_values):
    adj_p = jnp.pad(adj_indices, ((0, 0), (0, EP - E)))
    vals_p = jnp.pad(adj_values, (0, EP - E))
    return adj_p, vals_p


def _split_spec(bm):
    return pl.BlockSpec((2, bm, HD), lambda i: (0, i, 0))


def _split_spec_off(bm, off):
    return pl.BlockSpec((2, bm, HD), lambda i: (0, i + off, 0))


def kernel(adj_indices, adj_values, user_id, item_tid, item_iid, upref_img,
           upref_txt, image_feats, text_feats, t2i_W, t2i_b, i2i_W, i2i_b,
           ti_W1, ti_b1, ti_w2, ii_W1, ii_b1, ii_w2, fi_W1, fi_b1, fi_w2,
           uf_W1, uf_b1, uf_w2, itf_W1, itf_b1, itf_w2):
    f32 = jnp.float32
    vec = lambda v: v.reshape(1, D)

    n_i = I // BM_I
    items_out = pl.pallas_call(
        _items_dense_kernel,
        grid=(n_i,),
        in_specs=[
            pl.BlockSpec((BM_I, 4096), lambda i: (i, 0)),
            pl.BlockSpec((BM_I, 768), lambda i: (i, 0)),
            _row_spec(BM_I), _row_spec(BM_I),
            _full_spec((4096, D)), _full_spec((1, D)),
            _full_spec((768, D)), _full_spec((1, D)),
            _full_spec((D, D)), _full_spec((1, D)), _full_spec((1, D)),
            _full_spec((D, D)), _full_spec((1, D)), _full_spec((1, D)),
            _full_spec((D, D)), _full_spec((1, D)), _full_spec((1, D)),
        ],
        out_specs=[_row_spec(BM_I)] * 5 + [_split_spec(BM_I)] * 4,
        out_shape=[jax.ShapeDtypeStruct((I, D), f32)] * 5
        + [jax.ShapeDtypeStruct((2, I, HD), f32)] * 4,
    )(image_feats, text_feats, item_tid, item_iid,
      i2i_W.T, vec(i2i_b), t2i_W.T, vec(t2i_b),
      ii_W1.T, vec(ii_b1), vec(ii_w2),
      ti_W1.T, vec(ti_b1), vec(ti_w2),
      fi_W1.T, vec(fi_b1), vec(fi_w2))
    image, text, isi, ist, ife, imgn_s, txtn_s, guii_s, guti_s = items_out

    n_u = U // BM_U
    uimgn_s, utxtn_s, guu_s = pl.pallas_call(
        _users_norm_kernel,
        grid=(n_u,),
        in_specs=[_row_spec(BM_U)] * 3,
        out_specs=[_split_spec(BM_U)] * 3,
        out_shape=[jax.ShapeDtypeStruct((2, U, HD), f32)] * 3,
    )(upref_img, upref_txt, user_id)

    pad_z = jnp.zeros((2, NP - N, HD), f32)
    x0_img = jnp.concatenate([uimgn_s, imgn_s, pad_z], axis=1)
    x0_txt = jnp.concatenate([utxtn_s, txtn_s, pad_z], axis=1)
    gu_ii = jnp.concatenate([guu_s, guii_s, pad_z], axis=1)
    gu_ti = jnp.concatenate([guu_s, guti_s, pad_z], axis=1)

    adj_p, vals_p = _pad_edges(adj_indices, adj_values)
    _e1i, esum_img = _spmm2_sc(adj_p, vals_p, x0_img, gu_ii)
    _e1t, esum_txt = _spmm2_sc(adj_p, vals_p, x0_txt, gu_ti)

    u_g = pl.pallas_call(
        _final_users_kernel,
        grid=(n_u,),
        in_specs=[_split_spec(BM_U)] * 2 + [
            _full_spec((D, D)), _full_spec((1, D)), _full_spec((1, D))],
        out_specs=_row_spec(BM_U),
        out_shape=jax.ShapeDtypeStruct((U, D), f32),
    )(esum_txt, esum_img, uf_W1.T, vec(uf_b1), vec(uf_w2))

    i_g = pl.pallas_call(
        _final_items_kernel,
        grid=(n_i,),
        in_specs=[_split_spec_off(BM_I, U // BM_I)] * 2 + [
            _row_spec(BM_I),
            _full_spec((D, D)), _full_spec((1, D)), _full_spec((1, D))],
        out_specs=_row_spec(BM_I),
        out_shape=jax.ShapeDtypeStruct((I, D), f32),
    )(esum_txt, esum_img, ife, itf_W1.T, vec(itf_b1), vec(itf_w2))

    return (u_g, i_g, image, item_iid, text, item_tid, isi, ist, ife)


# speedup vs baseline: 1.0820x; 1.0301x over previous
import functools

import jax
import jax.numpy as jnp
from jax import lax
from jax.experimental import pallas as pl
from jax.experimental.pallas import tpu as pltpu
from jax.experimental.pallas import tpu_sc as plsc

U = 30000
I = 20000
N = U + I
D = 64
E = 800000
GAMMA = 0.5
N_LAYERS = 2

BM_I = 1000
BM_U = 1000


def _ets_block(x1, x2, W1t, b1, w2):
    q1 = jnp.dot(jnp.tanh(jnp.dot(x1, W1t, preferred_element_type=jnp.float32) + b1), w2,
                 preferred_element_type=jnp.float32)
    q2 = jnp.dot(jnp.tanh(jnp.dot(x2, W1t, preferred_element_type=jnp.float32) + b1), w2,
                 preferred_element_type=jnp.float32)
    w0 = jax.nn.sigmoid(q1 - q2)[:, None]
    return w0 * x1 + (1.0 - w0) * x2


def _normalize_block(x):
    n = jnp.sqrt(jnp.sum(x * x, axis=1, keepdims=True))
    return x / jnp.maximum(n, 1e-12)


def _store_split(out_ref, x):
    out_ref[0] = x[:, :32]
    out_ref[1] = x[:, 32:]


def _load_split(ref):
    return jnp.concatenate([ref[0], ref[1]], axis=1)


def _items_dense_kernel(imgf_ref, txtf_ref, itid_ref, iiid_ref,
                        i2iWt_ref, i2ib_ref, t2iWt_ref, t2ib_ref,
                        iiW1t_ref, iib1_ref, iiw2_ref,
                        tiW1t_ref, tib1_ref, tiw2_ref,
                        fiW1t_ref, fib1_ref, fiw2_ref,
                        image_out, text_out, isi_out, ist_out, ife_out,
                        imgn_out, txtn_out, guii_out, guti_out):
    image = jnp.dot(imgf_ref[...], i2iWt_ref[...],
                    preferred_element_type=jnp.float32) + i2ib_ref[...]
    text = jnp.dot(txtf_ref[...], t2iWt_ref[...],
                   preferred_element_type=jnp.float32) + t2ib_ref[...]
    isi = _ets_block(image, iiid_ref[...], iiW1t_ref[...], iib1_ref[...],
                     iiw2_ref[0, :])
    ist = _ets_block(text, itid_ref[...], tiW1t_ref[...], tib1_ref[...],
                     tiw2_ref[0, :])
    ife = _ets_block(isi, ist, fiW1t_ref[...], fib1_ref[...], fiw2_ref[0, :])
    image_out[...] = image
    text_out[...] = text
    isi_out[...] = isi
    ist_out[...] = ist
    ife_out[...] = ife
    _store_split(imgn_out, _normalize_block(image))
    _store_split(txtn_out, _normalize_block(text))
    _store_split(guii_out, GAMMA * iiid_ref[...])
    _store_split(guti_out, GAMMA * itid_ref[...])


def _users_norm_kernel(uimg_ref, utxt_ref, uid_ref,
                       uimgn_out, utxtn_out, guu_out):
    _store_split(uimgn_out, _normalize_block(uimg_ref[...]))
    _store_split(utxtn_out, _normalize_block(utxt_ref[...]))
    _store_split(guu_out, GAMMA * uid_ref[...])


def _final_users_kernel(t0_ref, t1_ref, t2_ref, g0_ref, g1_ref, g2_ref,
                        W1t_ref, b1_ref, w2_ref, ug_out):
    all_txt = (_load_split(t0_ref) + _load_split(t1_ref)
               + _load_split(t2_ref)) * (1.0 / 3.0)
    all_img = (_load_split(g0_ref) + _load_split(g1_ref)
               + _load_split(g2_ref)) * (1.0 / 3.0)
    ug_out[...] = _ets_block(all_txt, all_img, W1t_ref[...], b1_ref[...],
                             w2_ref[0, :])


def _final_items_kernel(t0_ref, t1_ref, t2_ref, g0_ref, g1_ref, g2_ref,
                        ife_ref, W1t_ref, b1_ref, w2_ref, ig_out):
    all_txt = (_load_split(t0_ref) + _load_split(t1_ref)
               + _load_split(t2_ref)) * (1.0 / 3.0)
    all_img = (_load_split(g0_ref) + _load_split(g1_ref)
               + _load_split(g2_ref)) * (1.0 / 3.0)
    ig_out[...] = (_ets_block(all_txt, all_img, W1t_ref[...], b1_ref[...],
                              w2_ref[0, :])
                   + _normalize_block(ife_ref[...]))


def _row_spec(bm):
    return pl.BlockSpec((bm, D), lambda i: (i, 0))


def _full_spec(shape):
    return pl.BlockSpec(shape, lambda i: tuple(0 for _ in shape))


NSUB = 16
HD = D // 2
KE = 128
NP = 50048
RPW = NP // NSUB
NCHUNK = 394
EPW = NCHUNK * KE
EP = EPW * NSUB


def _spmm2_sc_body(adj_hbm, vals_hbm, x0_hbm, gu_hbm, e1_hbm, e2_hbm,
                   acc, eidx, vv, xr,
                   gs0, gs1, ss0, ss1, es0, es1, es2):
    cid = lax.axis_index("c")
    sid = lax.axis_index("s")
    rlo = sid * RPW
    gs = (gs0, gs1)
    ss = (ss0, ss1)
    es = (es0, es1, es2)

    def init_acc(gu_hbm):
        pltpu.sync_copy(gu_hbm.at[cid, pl.ds(rlo, RPW)],
                        acc.at[pl.ds(rlo, RPW)])

    def edge_pass(src_hbm):
        src = src_hbm.at[cid]
        ebase = sid * EPW

        def issue_edata(c, b3):
            pltpu.async_copy(adj_hbm.at[:, pl.ds(ebase + c * KE, KE)],
                             eidx.at[b3], es[b3])
            pltpu.async_copy(vals_hbm.at[pl.ds(ebase + c * KE, KE)],
                             vv.at[b3], es[b3])

        def wait_edata(b3):
            pltpu.make_async_copy(adj_hbm.at[:, pl.ds(0, KE)],
                                  eidx.at[b3], es[b3]).wait()
            pltpu.make_async_copy(vals_hbm.at[pl.ds(0, KE)],
                                  vv.at[b3], es[b3]).wait()

        def issue_gather(b3, b2):
            pltpu.async_copy(src.at[eidx.at[b3].at[1]], xr.at[b2], gs[b2])

        def wait_gather(b2):
            pltpu.make_async_copy(src.at[eidx.at[0].at[1]],
                                  xr.at[b2], gs[b2]).wait()

        def issue_scatter(b3, b2):
            pltpu.async_copy(xr.at[b2], acc.at[eidx.at[b3].at[0]], ss[b2],
                             add=True)

        def wait_scatter(b2):
            pltpu.make_async_copy(xr.at[b2], acc.at[eidx.at[0].at[0]],
                                  ss[b2]).wait()

        def scale(b3, b2):
            def grp(g16, _):
                w = vv.at[b3][pl.ds(g16 * 16, 16)]
                e0 = g16 * 16
                for j in range(16):
                    v = w[j]
                    xr[b2, e0 + j, pl.ds(0, 16)] = \
                        xr[b2, e0 + j, pl.ds(0, 16)] * v
                    xr[b2, e0 + j, pl.ds(16, 16)] = \
                        xr[b2, e0 + j, pl.ds(16, 16)] * v
                return 0

            lax.fori_loop(0, KE // 16, grp, 0)

        def slot(c, b2, b3, first=False, do_next_gather=True,
                 do_edata=True):
            nb2, nb3 = 1 - b2, (b3 + 1) % 3
            if do_next_gather:
                wait_edata(nb3)
                if not first:
                    wait_scatter(nb2)
                issue_gather(nb3, nb2)
            if do_edata:
                issue_edata(c + 2, (b3 + 2) % 3)
            wait_gather(b2)
            scale(b3, b2)
            issue_scatter(b3, b2)

        issue_edata(0, 0)
        issue_edata(1, 1)
        wait_edata(0)
        issue_gather(0, 0)
        slot(0, 0, 0, first=True)
        slot(1, 1, 1)

        def body(k, _):
            c0 = 2 + 6 * k
            for j in range(6):
                slot(c0 + j, j % 2, (2 + j) % 3)
            return 0

        lax.fori_loop(0, (NCHUNK - 4) // 6, body, 0)
        slot(NCHUNK - 2, 0, (NCHUNK - 2) % 3, do_edata=False)
        slot(NCHUNK - 1, 1, (NCHUNK - 1) % 3, do_next_gather=False,
             do_edata=False)
        wait_scatter(0)
        wait_scatter(1)

    init_acc(gu_hbm)
    plsc.subcore_barrier()
    edge_pass(x0_hbm)
    plsc.subcore_barrier()
    pltpu.sync_copy(acc.at[pl.ds(rlo, RPW)],
                    e1_hbm.at[cid, pl.ds(rlo, RPW)])
    plsc.subcore_barrier()
    init_acc(gu_hbm)
    plsc.subcore_barrier()
    edge_pass(e1_hbm)
    plsc.subcore_barrier()
    pltpu.sync_copy(acc.at[pl.ds(rlo, RPW)],
                    e2_hbm.at[cid, pl.ds(rlo, RPW)])


@functools.lru_cache(maxsize=1)
def _get_spmm2_sc():
    return pl.kernel(
        _spmm2_sc_body,
        mesh=plsc.VectorSubcoreMesh(core_axis_name="c",
                                    subcore_axis_name="s"),
        compiler_params=pltpu.CompilerParams(use_tc_tiling_on_sc=False),
        out_type=[jax.ShapeDtypeStruct((2, NP, HD), jnp.float32)] * 2,
        scratch_types=[
            pltpu.VMEM_SHARED((NP, HD), jnp.float32),
            pltpu.VMEM((3, 2, KE), jnp.int32),
            pltpu.VMEM((3, KE), jnp.float32),
            pltpu.VMEM((2, KE, HD), jnp.float32),
        ] + [pltpu.SemaphoreType.DMA] * 7,
    )


def _spmm2_sc(*args):
    return _get_spmm2_sc()(*args)


def _pad_edges(adj_indices, adj_values):
    adj_p = jnp.pad(adj_indices, ((0, 0), (0, EP - E)))
    vals_p = jnp.pad(adj_values, (0, EP - E))
    return adj_p, vals_p


def _split_spec(bm):
    return pl.BlockSpec((2, bm, HD), lambda i: (0, i, 0))


def _split_spec_off(bm, off):
    return pl.BlockSpec((2, bm, HD), lambda i: (0, i + off, 0))


def kernel(adj_indices, adj_values, user_id, item_tid, item_iid, upref_img,
           upref_txt, image_feats, text_feats, t2i_W, t2i_b, i2i_W, i2i_b,
           ti_W1, ti_b1, ti_w2, ii_W1, ii_b1, ii_w2, fi_W1, fi_b1, fi_w2,
           uf_W1, uf_b1, uf_w2, itf_W1, itf_b1, itf_w2):
    f32 = jnp.float32
    vec = lambda v: v.reshape(1, D)

    n_i = I // BM_I
    items_out = pl.pallas_call(
        _items_dense_kernel,
        grid=(n_i,),
        in_specs=[
            pl.BlockSpec((BM_I, 4096), lambda i: (i, 0)),
            pl.BlockSpec((BM_I, 768), lambda i: (i, 0)),
            _row_spec(BM_I), _row_spec(BM_I),
            _full_spec((4096, D)), _full_spec((1, D)),
            _full_spec((768, D)), _full_spec((1, D)),
            _full_spec((D, D)), _full_spec((1, D)), _full_spec((1, D)),
            _full_spec((D, D)), _full_spec((1, D)), _full_spec((1, D)),
            _full_spec((D, D)), _full_spec((1, D)), _full_spec((1, D)),
        ],
        out_specs=[_row_spec(BM_I)] * 5 + [_split_spec(BM_I)] * 4,
        out_shape=[jax.ShapeDtypeStruct((I, D), f32)] * 5
        + [jax.ShapeDtypeStruct((2, I, HD), f32)] * 4,
    )(image_feats, text_feats, item_tid, item_iid,
      i2i_W.T, vec(i2i_b), t2i_W.T, vec(t2i_b),
      ii_W1.T, vec(ii_b1), vec(ii_w2),
      ti_W1.T, vec(ti_b1), vec(ti_w2),
      fi_W1.T, vec(fi_b1), vec(fi_w2))
    image, text, isi, ist, ife, imgn_s, txtn_s, guii_s, guti_s = items_out

    n_u = U // BM_U
    uimgn_s, utxtn_s, guu_s = pl.pallas_call(
        _users_norm_kernel,
        grid=(n_u,),
        in_specs=[_row_spec(BM_U)] * 3,
        out_specs=[_split_spec(BM_U)] * 3,
        out_shape=[jax.ShapeDtypeStruct((2, U, HD), f32)] * 3,
    )(upref_img, upref_txt, user_id)

    pad_z = jnp.zeros((2, NP - N, HD), f32)
    x0_img = jnp.concatenate([uimgn_s, imgn_s, pad_z], axis=1)
    x0_txt = jnp.concatenate([utxtn_s, txtn_s, pad_z], axis=1)
    gu_ii = jnp.concatenate([guu_s, guii_s, pad_z], axis=1)
    gu_ti = jnp.concatenate([guu_s, guti_s, pad_z], axis=1)

    adj_p, vals_p = _pad_edges(adj_indices, adj_values)
    img1, img2 = _spmm2_sc(adj_p, vals_p, x0_img, gu_ii)
    txt1, txt2 = _spmm2_sc(adj_p, vals_p, x0_txt, gu_ti)

    u_g = pl.pallas_call(
        _final_users_kernel,
        grid=(n_u,),
        in_specs=[_split_spec(BM_U)] * 6 + [
            _full_spec((D, D)), _full_spec((1, D)), _full_spec((1, D))],
        out_specs=_row_spec(BM_U),
        out_shape=jax.ShapeDtypeStruct((U, D), f32),
    )(x0_txt, txt1, txt2, x0_img, img1, img2,
      uf_W1.T, vec(uf_b1), vec(uf_w2))

    i_g = pl.pallas_call(
        _final_items_kernel,
        grid=(n_i,),
        in_specs=[_split_spec_off(BM_I, U // BM_I)] * 6 + [
            _row_spec(BM_I),
            _full_spec((D, D)), _full_spec((1, D)), _full_spec((1, D))],
        out_specs=_row_spec(BM_I),
        out_shape=jax.ShapeDtypeStruct((I, D), f32),
    )(x0_txt, txt1, txt2, x0_img, img1, img2,
      ife, itf_W1.T, vec(itf_b1), vec(itf_w2))

    return (u_g, i_g, image, item_iid, text, item_tid, isi, ist, ife)
